# baseline (speedup 1.0000x reference)
import functools

import jax
import jax.numpy as jnp
from jax import lax
from jax.experimental import pallas as pl
from jax.experimental.pallas import tpu as pltpu
from jax.experimental.pallas import tpu_sc as plsc

NC = 2
NS = 16
NW = NC * NS

V = 512
D = 256
B = 16384

_CHUNK = 128
_A_CHUNKS = (V * D) // _CHUNK // NW
_B_ROWS = B // NW
_B_CHUNKS = _B_ROWS // _CHUNK


def _mesh():
    return plsc.VectorSubcoreMesh(core_axis_name="c", subcore_axis_name="s")


def _diag_body(t_hbm, sidx_hbm, diag_hbm, sidx_v, svals_v, gsem):
    wid = lax.axis_index("s") * NC + lax.axis_index("c")
    r0 = wid * _A_CHUNKS
    pltpu.sync_copy(sidx_hbm.at[pl.ds(r0, _A_CHUNKS)], sidx_v)
    handles = []
    for k in range(_A_CHUNKS):
        handles.append(
            pltpu.async_copy(t_hbm.at[sidx_v.at[k]], svals_v.at[k], gsem))
    for h in handles:
        h.wait()
    pltpu.sync_copy(svals_v, diag_hbm.at[pl.ds(r0, _A_CHUNKS)])


def _gather_body(diag_hbm, idx_hbm, out_hbm, bidx_v, rows_v, gsem0, gsem1):
    wid = lax.axis_index("s") * NC + lax.axis_index("c")
    base = wid * _B_ROWS
    pltpu.sync_copy(idx_hbm.at[pl.ds(wid * _B_CHUNKS, _B_CHUNKS)], bidx_v)
    sems = (gsem0, gsem1)
    h = pltpu.async_copy(diag_hbm.at[bidx_v.at[0]], rows_v.at[0], sems[0])
    for j in range(_B_CHUNKS):
        h.wait()
        if j + 1 < _B_CHUNKS:
            nb = (j + 1) % 2
            h = pltpu.async_copy(
                diag_hbm.at[bidx_v.at[j + 1]], rows_v.at[nb], sems[nb])
        pltpu.sync_copy(rows_v.at[j % 2],
                        out_hbm.at[pl.ds(base + j * _CHUNK, _CHUNK)])


@functools.partial(
    pl.kernel,
    out_type=jax.ShapeDtypeStruct((V * D // _CHUNK, _CHUNK), jnp.float32),
    mesh=_mesh(),
    scratch_types=[
        pltpu.VMEM((_A_CHUNKS, _CHUNK), jnp.int32),
        pltpu.VMEM((_A_CHUNKS, _CHUNK), jnp.float32),
        pltpu.SemaphoreType.DMA,
    ],
)
def _diag_kernel(t_hbm, sidx_hbm, diag_hbm, sidx_v, svals_v, gsem):
    _diag_body(t_hbm, sidx_hbm, diag_hbm, sidx_v, svals_v, gsem)


@functools.partial(
    pl.kernel,
    out_type=jax.ShapeDtypeStruct((B, D), jnp.float32),
    mesh=_mesh(),
    scratch_types=[
        pltpu.VMEM((_B_CHUNKS, _CHUNK), jnp.int32),
        pltpu.VMEM((2, _CHUNK, D), jnp.float32),
        pltpu.SemaphoreType.DMA,
        pltpu.SemaphoreType.DMA,
    ],
)
def _gather_kernel(diag_hbm, idx_hbm, out_hbm, bidx_v, rows_v, gsem0, gsem1):
    _gather_body(diag_hbm, idx_hbm, out_hbm, bidx_v, rows_v, gsem0, gsem1)


def kernel(t, idx):
    t_flat = t.reshape(-1)
    idx2 = idx.astype(jnp.int32).reshape(B // _CHUNK, _CHUNK)
    vv = jnp.arange(V, dtype=jnp.int32)
    dd = jnp.arange(D, dtype=jnp.int32)
    sidx = (vv[:, None] * (D * V + 1) + dd[None, :] * V).reshape(
        V * D // _CHUNK, _CHUNK)
    diag = _diag_kernel(t_flat, sidx)
    diag = diag.reshape(V, D)
    return _gather_kernel(diag, idx2)


# speedup vs baseline: 2.6117x; 2.6117x over previous
import functools

import jax
import jax.numpy as jnp
from jax import lax
from jax.experimental import pallas as pl
from jax.experimental.pallas import tpu as pltpu
from jax.experimental.pallas import tpu_sc as plsc

NC = 2
NS = 16
NW = NC * NS

V = 512
D = 256
B = 16384

_CHUNK = 128
_A_ROWS = V // NW
_B_ROWS = B // NW
_B_CHUNKS = _B_ROWS // _CHUNK


def _mesh():
    return plsc.VectorSubcoreMesh(core_axis_name="c", subcore_axis_name="s")


_A_BLK = 8


def _diag_tc_body(blk_ref, out_ref):
    i = pl.program_id(0)
    c0 = (i % (128 // _A_BLK)) * _A_BLK
    row = lax.broadcasted_iota(jnp.int32, (_A_BLK, 1, 128), 0)
    lane = lax.broadcasted_iota(jnp.int32, (_A_BLK, 1, 128), 2)
    onehot = (lane == c0 + row).astype(jnp.float32)
    out_ref[...] = jnp.sum(blk_ref[...] * onehot, axis=2)


def _gather_body(diag_hbm, idx_hbm, out_hbm, bidx_v, rows_v, gsem0, gsem1):
    wid = lax.axis_index("s") * NC + lax.axis_index("c")
    base = wid * _B_ROWS
    pltpu.sync_copy(idx_hbm.at[pl.ds(wid * _B_CHUNKS, _B_CHUNKS)], bidx_v)
    sems = (gsem0, gsem1)
    h = pltpu.async_copy(diag_hbm.at[bidx_v.at[0]], rows_v.at[0], sems[0])
    for j in range(_B_CHUNKS):
        h.wait()
        if j + 1 < _B_CHUNKS:
            nb = (j + 1) % 2
            h = pltpu.async_copy(
                diag_hbm.at[bidx_v.at[j + 1]], rows_v.at[nb], sems[nb])
        pltpu.sync_copy(rows_v.at[j % 2],
                        out_hbm.at[pl.ds(base + j * _CHUNK, _CHUNK)])


_diag_kernel = pl.pallas_call(
    _diag_tc_body,
    grid=(V // _A_BLK,),
    in_specs=[pl.BlockSpec((_A_BLK, D, 128),
                           lambda i: (i, 0, i // (128 // _A_BLK)))],
    out_specs=pl.BlockSpec((_A_BLK, D), lambda i: (i, 0)),
    out_shape=jax.ShapeDtypeStruct((V, D), jnp.float32),
)


@functools.partial(
    pl.kernel,
    out_type=jax.ShapeDtypeStruct((B, D), jnp.float32),
    mesh=_mesh(),
    scratch_types=[
        pltpu.VMEM((_B_CHUNKS, _CHUNK), jnp.int32),
        pltpu.VMEM((2, _CHUNK, D), jnp.float32),
        pltpu.SemaphoreType.DMA,
        pltpu.SemaphoreType.DMA,
    ],
)
def _gather_kernel(diag_hbm, idx_hbm, out_hbm, bidx_v, rows_v, gsem0, gsem1):
    _gather_body(diag_hbm, idx_hbm, out_hbm, bidx_v, rows_v, gsem0, gsem1)


def kernel(t, idx):
    idx2 = idx.astype(jnp.int32).reshape(B // _CHUNK, _CHUNK)
    diag = _diag_kernel(t)
    return _gather_kernel(diag, idx2)


# speedup vs baseline: 3.0026x; 1.1497x over previous
import functools

import jax
import jax.numpy as jnp
from jax import lax
from jax.experimental import pallas as pl
from jax.experimental.pallas import tpu as pltpu
from jax.experimental.pallas import tpu_sc as plsc

NC = 2
NS = 16
NW = NC * NS

V = 512
D = 256
B = 16384

_CHUNK = 128
_A_ROWS = V // NW
_B_ROWS = B // NW
_B_CHUNKS = _B_ROWS // _CHUNK


def _mesh():
    return plsc.VectorSubcoreMesh(core_axis_name="c", subcore_axis_name="s")


_A_ROWS = V // NW
_LANES = 16


def _diag_body(t_hbm, diag_hbm, win0_v, win1_v, dblk_v, sem0, sem1, dsem):
    wid = lax.axis_index("s") * NC + lax.axis_index("c")
    v0 = wid * _A_ROWS
    w0 = (wid // (128 // _A_ROWS)) * 128
    bufs = (win0_v, win1_v)
    sems = (sem0, sem1)

    def win_copy(j):
        return pltpu.async_copy(
            t_hbm.at[v0 + j, :, pl.ds(w0, 128)], bufs[j % 2], sems[j % 2])

    lane = lax.iota(jnp.int32, _LANES)
    h = win_copy(0)
    for j in range(_A_ROWS):
        h.wait()
        if j + 1 < _A_ROWS:
            h = win_copy(j + 1)
        col = jnp.full((_LANES,), (wid % (128 // _A_ROWS)) * _A_ROWS + j,
                       dtype=jnp.int32)
        for k in range(D // _LANES):
            rows = k * _LANES + lane
            vals = plsc.load_gather(bufs[j % 2], [rows, col])
            dblk_v[pl.ds(j * D + k * _LANES, _LANES)] = vals
    handles = []
    for j in range(_A_ROWS):
        handles.append(pltpu.async_copy(
            dblk_v.at[pl.ds(j * D, D)], diag_hbm.at[v0 + j], dsem))
    for h in handles:
        h.wait()


def _gather_body(diag_hbm, idx_hbm, out_hbm, bidx_v, rows_v, gsem0, gsem1):
    wid = lax.axis_index("s") * NC + lax.axis_index("c")
    base = wid * _B_ROWS
    pltpu.sync_copy(idx_hbm.at[pl.ds(wid * _B_CHUNKS, _B_CHUNKS)], bidx_v)
    sems = (gsem0, gsem1)
    h = pltpu.async_copy(diag_hbm.at[bidx_v.at[0]], rows_v.at[0], sems[0])
    for j in range(_B_CHUNKS):
        h.wait()
        if j + 1 < _B_CHUNKS:
            nb = (j + 1) % 2
            h = pltpu.async_copy(
                diag_hbm.at[bidx_v.at[j + 1]], rows_v.at[nb], sems[nb])
        pltpu.sync_copy(rows_v.at[j % 2],
                        out_hbm.at[pl.ds(base + j * _CHUNK, _CHUNK)])


@functools.partial(
    pl.kernel,
    out_type=jax.ShapeDtypeStruct((V, D), jnp.float32),
    mesh=_mesh(),
    scratch_types=[
        pltpu.VMEM((D, 128), jnp.float32),
        pltpu.VMEM((D, 128), jnp.float32),
        pltpu.VMEM((_A_ROWS * D,), jnp.float32),
        pltpu.SemaphoreType.DMA,
        pltpu.SemaphoreType.DMA,
        pltpu.SemaphoreType.DMA,
    ],
    compiler_params=pltpu.CompilerParams(needs_layout_passes=False),
)
def _diag_kernel(t_hbm, diag_hbm, win0_v, win1_v, dblk_v, sem0, sem1, dsem):
    _diag_body(t_hbm, diag_hbm, win0_v, win1_v, dblk_v, sem0, sem1, dsem)


@functools.partial(
    pl.kernel,
    out_type=jax.ShapeDtypeStruct((B, D), jnp.float32),
    mesh=_mesh(),
    scratch_types=[
        pltpu.VMEM((_B_CHUNKS, _CHUNK), jnp.int32),
        pltpu.VMEM((2, _CHUNK, D), jnp.float32),
        pltpu.SemaphoreType.DMA,
        pltpu.SemaphoreType.DMA,
    ],
)
def _gather_kernel(diag_hbm, idx_hbm, out_hbm, bidx_v, rows_v, gsem0, gsem1):
    _gather_body(diag_hbm, idx_hbm, out_hbm, bidx_v, rows_v, gsem0, gsem1)


def kernel(t, idx):
    idx2 = idx.astype(jnp.int32).reshape(B // _CHUNK, _CHUNK)
    diag = _diag_kernel(t)
    return _gather_kernel(diag, idx2)


# speedup vs baseline: 3.5229x; 1.1733x over previous
import functools

import jax
import jax.numpy as jnp
from jax import lax
from jax.experimental import pallas as pl
from jax.experimental.pallas import tpu as pltpu
from jax.experimental.pallas import tpu_sc as plsc

NC = 2
NS = 16
NW = NC * NS

V = 512
D = 256
B = 16384

V_TC = 192
V_SC = V - V_TC

_CHUNK = 128
_LANES = 16
_A_BLK = 8
_A_SC_ROWS = V_SC // NW
_B_ROWS = B // NW
_B_CHUNKS = _B_ROWS // _CHUNK


def _mesh():
    return plsc.VectorSubcoreMesh(core_axis_name="c", subcore_axis_name="s")


def _diag_tc_body(blk_ref, out_ref):
    i = pl.program_id(0)
    c0 = (i % (128 // _A_BLK)) * _A_BLK
    row = lax.broadcasted_iota(jnp.int32, (_A_BLK, 1, 128), 0)
    lane = lax.broadcasted_iota(jnp.int32, (_A_BLK, 1, 128), 2)
    onehot = (lane == c0 + row).astype(jnp.float32)
    out_ref[...] = jnp.sum(blk_ref[...] * onehot, axis=2)


_diag_tc_kernel = pl.pallas_call(
    _diag_tc_body,
    grid=(V_TC // _A_BLK,),
    in_specs=[pl.BlockSpec((_A_BLK, D, 128),
                           lambda i: (i, 0, i // (128 // _A_BLK)))],
    out_specs=pl.BlockSpec((_A_BLK, D), lambda i: (i, 0)),
    out_shape=jax.ShapeDtypeStruct((V_TC, D), jnp.float32),
)


def _diag_sc_body(t_hbm, diag_hbm, win0_v, win1_v, dblk_v, sem0, sem1, dsem):
    wid = lax.axis_index("s") * NC + lax.axis_index("c")
    v0 = V_TC + wid * _A_SC_ROWS
    bufs = (win0_v, win1_v)
    sems = (sem0, sem1)

    def win_copy(j):
        w0 = pl.multiple_of(((v0 + j) // 128) * 128, 128)
        return pltpu.async_copy(
            t_hbm.at[v0 + j, :, pl.ds(w0, 128)], bufs[j % 2], sems[j % 2])

    lane = lax.iota(jnp.int32, _LANES)
    h = win_copy(0)
    for j in range(_A_SC_ROWS):
        h.wait()
        if j + 1 < _A_SC_ROWS:
            h = win_copy(j + 1)
        col = jnp.full((_LANES,), (v0 + j) % 128, dtype=jnp.int32)
        for k in range(D // _LANES):
            rows = k * _LANES + lane
            vals = plsc.load_gather(bufs[j % 2], [rows, col])
            dblk_v[pl.ds(j * D + k * _LANES, _LANES)] = vals
    handles = []
    for j in range(_A_SC_ROWS):
        handles.append(pltpu.async_copy(
            dblk_v.at[pl.ds(j * D, D)], diag_hbm.at[wid * _A_SC_ROWS + j],
            dsem))
    for h in handles:
        h.wait()


@functools.partial(
    pl.kernel,
    out_type=jax.ShapeDtypeStruct((V_SC, D), jnp.float32),
    mesh=_mesh(),
    scratch_types=[
        pltpu.VMEM((D, 128), jnp.float32),
        pltpu.VMEM((D, 128), jnp.float32),
        pltpu.VMEM((_A_SC_ROWS * D,), jnp.float32),
        pltpu.SemaphoreType.DMA,
        pltpu.SemaphoreType.DMA,
        pltpu.SemaphoreType.DMA,
    ],
    compiler_params=pltpu.CompilerParams(needs_layout_passes=False),
)
def _diag_sc_kernel(t_hbm, diag_hbm, win0_v, win1_v, dblk_v, sem0, sem1,
                    dsem):
    _diag_sc_body(t_hbm, diag_hbm, win0_v, win1_v, dblk_v, sem0, sem1, dsem)


def _gather_body(diag_hbm, idx_hbm, out_hbm, bidx_v, rows_v, gsem0, gsem1):
    wid = lax.axis_index("s") * NC + lax.axis_index("c")
    base = wid * _B_ROWS
    pltpu.sync_copy(idx_hbm.at[pl.ds(wid * _B_CHUNKS, _B_CHUNKS)], bidx_v)
    sems = (gsem0, gsem1)
    h = pltpu.async_copy(diag_hbm.at[bidx_v.at[0]], rows_v.at[0], sems[0])
    for j in range(_B_CHUNKS):
        h.wait()
        if j + 1 < _B_CHUNKS:
            nb = (j + 1) % 2
            h = pltpu.async_copy(
                diag_hbm.at[bidx_v.at[j + 1]], rows_v.at[nb], sems[nb])
        pltpu.sync_copy(rows_v.at[j % 2],
                        out_hbm.at[pl.ds(base + j * _CHUNK, _CHUNK)])


@functools.partial(
    pl.kernel,
    out_type=jax.ShapeDtypeStruct((B, D), jnp.float32),
    mesh=_mesh(),
    scratch_types=[
        pltpu.VMEM((_B_CHUNKS, _CHUNK), jnp.int32),
        pltpu.VMEM((2, _CHUNK, D), jnp.float32),
        pltpu.SemaphoreType.DMA,
        pltpu.SemaphoreType.DMA,
    ],
)
def _gather_kernel(diag_hbm, idx_hbm, out_hbm, bidx_v, rows_v, gsem0, gsem1):
    _gather_body(diag_hbm, idx_hbm, out_hbm, bidx_v, rows_v, gsem0, gsem1)


def kernel(t, idx):
    idx2 = idx.astype(jnp.int32).reshape(B // _CHUNK, _CHUNK)
    diag_tc = _diag_tc_kernel(t)
    diag_sc = _diag_sc_kernel(t)
    diag = jnp.concatenate([diag_tc, diag_sc], axis=0)
    return _gather_kernel(diag, idx2)


# speedup vs baseline: 3.5310x; 1.0023x over previous
import functools

import jax
import jax.numpy as jnp
from jax import lax
from jax.experimental import pallas as pl
from jax.experimental.pallas import tpu as pltpu
from jax.experimental.pallas import tpu_sc as plsc

NC = 2
NS = 16
NW = NC * NS

V = 512
D = 256
B = 16384

V_TC = 192
V_SC = V - V_TC

_CHUNK = 128
_LANES = 16
_A_BLK = 8
_A_SC_ROWS = V_SC // NW
_B_ROWS = B // NW
_B_CHUNKS = _B_ROWS // _CHUNK


def _mesh():
    return plsc.VectorSubcoreMesh(core_axis_name="c", subcore_axis_name="s")


def _diag_tc_body(blk_ref, out_ref):
    i = pl.program_id(0)
    c0 = (i % (128 // _A_BLK)) * _A_BLK
    row = lax.broadcasted_iota(jnp.int32, (_A_BLK, 1, 128), 0)
    lane = lax.broadcasted_iota(jnp.int32, (_A_BLK, 1, 128), 2)
    onehot = (lane == c0 + row).astype(jnp.float32)
    out_ref[...] = jnp.sum(blk_ref[...] * onehot, axis=2)


_diag_tc_kernel = pl.pallas_call(
    _diag_tc_body,
    grid=(V_TC // _A_BLK,),
    in_specs=[pl.BlockSpec((_A_BLK, D, 128),
                           lambda i: (i, 0, i // (128 // _A_BLK)))],
    out_specs=pl.BlockSpec((_A_BLK, D), lambda i: (i, 0)),
    out_shape=jax.ShapeDtypeStruct((V_TC, D), jnp.float32),
)


def _diag_sc_body(t_hbm, diag_hbm, win0_v, win1_v, dblk_v, sem0, sem1, dsem):
    wid = lax.axis_index("s") * NC + lax.axis_index("c")
    v0 = V_TC + wid * _A_SC_ROWS
    bufs = (win0_v, win1_v)
    sems = (sem0, sem1)

    def win_copy(j):
        w0 = pl.multiple_of(((v0 + j) // 128) * 128, 128)
        return pltpu.async_copy(
            t_hbm.at[v0 + j, :, pl.ds(w0, 128)], bufs[j % 2], sems[j % 2])

    lane = lax.iota(jnp.int32, _LANES)
    h = win_copy(0)
    for j in range(_A_SC_ROWS):
        h.wait()
        if j + 1 < _A_SC_ROWS:
            h = win_copy(j + 1)
        col = jnp.full((_LANES,), (v0 + j) % 128, dtype=jnp.int32)
        for k in range(D // _LANES):
            rows = k * _LANES + lane
            vals = plsc.load_gather(bufs[j % 2], [rows, col])
            dblk_v[pl.ds(j * D + k * _LANES, _LANES)] = vals
    handles = []
    for j in range(_A_SC_ROWS):
        handles.append(pltpu.async_copy(
            dblk_v.at[pl.ds(j * D, D)], diag_hbm.at[wid * _A_SC_ROWS + j],
            dsem))
    for h in handles:
        h.wait()


@functools.partial(
    pl.kernel,
    out_type=jax.ShapeDtypeStruct((V_SC, D), jnp.float32),
    mesh=_mesh(),
    scratch_types=[
        pltpu.VMEM((D, 128), jnp.float32),
        pltpu.VMEM((D, 128), jnp.float32),
        pltpu.VMEM((_A_SC_ROWS * D,), jnp.float32),
        pltpu.SemaphoreType.DMA,
        pltpu.SemaphoreType.DMA,
        pltpu.SemaphoreType.DMA,
    ],
    compiler_params=pltpu.CompilerParams(needs_layout_passes=False),
)
def _diag_sc_kernel(t_hbm, diag_hbm, win0_v, win1_v, dblk_v, sem0, sem1,
                    dsem):
    _diag_sc_body(t_hbm, diag_hbm, win0_v, win1_v, dblk_v, sem0, sem1, dsem)


def _merge_body(tc_ref, sc_ref, out_ref):
    out_ref[0:V_TC, :] = tc_ref[...]
    out_ref[V_TC:V, :] = sc_ref[...]


_merge_kernel = pl.pallas_call(
    _merge_body,
    out_shape=jax.ShapeDtypeStruct((V, D), jnp.float32),
)


def _gather_body(diag_hbm, idx_hbm, out_hbm, bidx_v, rows_v, gsem0, gsem1):
    wid = lax.axis_index("s") * NC + lax.axis_index("c")
    base = wid * _B_ROWS
    pltpu.sync_copy(idx_hbm.at[pl.ds(wid * _B_CHUNKS, _B_CHUNKS)], bidx_v)
    sems = (gsem0, gsem1)
    h = pltpu.async_copy(diag_hbm.at[bidx_v.at[0]], rows_v.at[0], sems[0])
    for j in range(_B_CHUNKS):
        h.wait()
        if j + 1 < _B_CHUNKS:
            nb = (j + 1) % 2
            h = pltpu.async_copy(
                diag_hbm.at[bidx_v.at[j + 1]], rows_v.at[nb], sems[nb])
        pltpu.sync_copy(rows_v.at[j % 2],
                        out_hbm.at[pl.ds(base + j * _CHUNK, _CHUNK)])


@functools.partial(
    pl.kernel,
    out_type=jax.ShapeDtypeStruct((B, D), jnp.float32),
    mesh=_mesh(),
    scratch_types=[
        pltpu.VMEM((_B_CHUNKS, _CHUNK), jnp.int32),
        pltpu.VMEM((2, _CHUNK, D), jnp.float32),
        pltpu.SemaphoreType.DMA,
        pltpu.SemaphoreType.DMA,
    ],
)
def _gather_kernel(diag_hbm, idx_hbm, out_hbm, bidx_v, rows_v, gsem0, gsem1):
    _gather_body(diag_hbm, idx_hbm, out_hbm, bidx_v, rows_v, gsem0, gsem1)


def kernel(t, idx):
    idx2 = idx.astype(jnp.int32).reshape(B // _CHUNK, _CHUNK)
    diag_tc = _diag_tc_kernel(t)
    diag_sc = _diag_sc_kernel(t)
    diag = _merge_kernel(diag_tc, diag_sc)
    return _gather_kernel(diag, idx2)
